# baseline (speedup 1.0000x reference)
import functools

import jax
import jax.numpy as jnp
from jax import lax
from jax.experimental import pallas as pl
from jax.experimental.pallas import tpu as pltpu
from jax.experimental.pallas import tpu_sc as plsc

NC = 2
NS = 16
NW = NC * NS
CHUNK = 128
K = 20


def _round_up(a, b):
    return (a + b - 1) // b * b


def _sc_mesh():
    return plsc.VectorSubcoreMesh(core_axis_name="c", subcore_axis_name="s",
                                  num_cores=NC, num_subcores=NS)


_SC_PARAMS = pltpu.CompilerParams(use_tc_tiling_on_sc=False)


def _worker_range(r0, r1, c, s):
    base = jnp.where(c == 0, s * r0, NS * r0 + s * r1)
    ngrp = jnp.where(c == 0, r0 // (2 * K), r1 // (2 * K))
    return base, ngrp


def _deg_body(r0, r1, tpt, t, dstr, ones_h, z1, degp, deg_sh, ones_v,
              didx_a, didx_b, ssem, zb):
    c = lax.axis_index("c")
    s = lax.axis_index("s")
    pltpu.sync_copy(z1, zb)
    pltpu.sync_copy(zb, deg_sh.at[pl.ds(s * tpt, tpt)])
    pltpu.sync_copy(ones_h, ones_v)
    plsc.subcore_barrier()
    base, ngrp = _worker_range(r0, r1, c, s)

    def grp(g2, carry):
        g = g2 * 2
        descs = []
        for didx, go in ((didx_a, g), (didx_b, g + 1)):
            pltpu.sync_copy(dstr.at[pl.ds(base + go * K, K)], didx)
            descs += [pltpu.async_copy(ones_v, deg_sh.at[didx.at[j]], ssem,
                                       add=True)
                      for j in range(K)]
        for d in descs:
            d.wait()
        return carry

    lax.fori_loop(0, ngrp, grp, 0)
    plsc.subcore_barrier()
    pltpu.sync_copy(deg_sh.at[pl.ds(s * tpt, tpt)], zb)
    pltpu.sync_copy(zb, degp.at[pl.ds(c * t + s * tpt, tpt)])


def _half_pass(r0, r1, tpt, c, s, u_h, srcr, dstr, z2, outp,
               tab_sh, bufs, gsem, ssem, zb):
    hpt = tpt // 2
    pltpu.sync_copy(z2, zb)
    for q in range(2):
        pltpu.sync_copy(zb, tab_sh.at[pl.ds(s * tpt + q * hpt, hpt)])
    plsc.subcore_barrier()
    base, ngrp = _worker_range(r0, r1, c, s)

    def grp(g2, carry):
        g = g2 * 2
        sdescs = []
        for (sidx, didx, rows), go in ((bufs[0], g), (bufs[1], g + 1)):
            pltpu.sync_copy(srcr.at[pl.ds(base + go * K, K)], sidx)
            pltpu.sync_copy(dstr.at[pl.ds(base + go * K, K)], didx)
            gdescs = [pltpu.async_copy(u_h.at[sidx.at[j]], rows.at[j], gsem)
                      for j in range(K)]
            for d in gdescs:
                d.wait()
            sdescs += [pltpu.async_copy(rows.at[j], tab_sh.at[didx.at[j]],
                                        ssem, add=True)
                       for j in range(K)]
        for d in sdescs:
            d.wait()
        return carry

    lax.fori_loop(0, ngrp, grp, 0)
    plsc.subcore_barrier()
    for q in range(2):
        pltpu.sync_copy(tab_sh.at[pl.ds(s * tpt + q * hpt, hpt)], zb)
        pltpu.sync_copy(zb, outp.at[c, pl.ds(s * tpt + q * hpt, hpt)])
    plsc.subcore_barrier()


def _layer1_body(r0, r1, tpt, ua, ub, srcr, dstr, z2, outa, outb,
                 tab_sh, sidx_a, didx_a, rows_a, sidx_b, didx_b, rows_b,
                 gsem, ssem, zb):
    c = lax.axis_index("c")
    s = lax.axis_index("s")
    bufs = ((sidx_a, didx_a, rows_a), (sidx_b, didx_b, rows_b))
    _half_pass(r0, r1, tpt, c, s, ua, srcr, dstr, z2, outa,
               tab_sh, bufs, gsem, ssem, zb)
    _half_pass(r0, r1, tpt, c, s, ub, srcr, dstr, z2, outb,
               tab_sh, bufs, gsem, ssem, zb)


def _layer2_body(r0, r1, tpt, u, srcr, dstr, z2, outp,
                 tab_sh, sidx_a, didx_a, rows_a, sidx_b, didx_b, rows_b,
                 gsem, ssem, zb):
    c = lax.axis_index("c")
    s = lax.axis_index("s")
    bufs = ((sidx_a, didx_a, rows_a), (sidx_b, didx_b, rows_b))
    _half_pass(r0, r1, tpt, c, s, u, srcr, dstr, z2, outp,
               tab_sh, bufs, gsem, ssem, zb)


def _dinv_packed(dp0, dp1, kf):
    deg = dp0[...] + dp1[...] + 1.0
    return jnp.dot(lax.rsqrt(deg), kf, preferred_element_type=jnp.float32)


def _dense1_body(dp0, dp1, xp, wa_r, wb_r, kf_r, ua_r, ub_r):
    dinvp = _dinv_packed(dp0, dp1, kf_r[...])
    ua_r[...] = jnp.dot(xp[...], wa_r[...],
                        preferred_element_type=jnp.float32) * dinvp
    ub_r[...] = jnp.dot(xp[...], wb_r[...],
                        preferred_element_type=jnp.float32) * dinvp


def _dense2_body(a0a, a1a, a0b, a1b, ua, ub, dp0, dp1, b1a_r, b1b_r,
                 w2a_r, w2b_r, kf_r, u2_r):
    dinvp = _dinv_packed(dp0, dp1, kf_r[...])
    ha = jnp.maximum(dinvp * (a0a[...] + a1a[...] + ua[...]) + b1a_r[...],
                     0.0)
    hb = jnp.maximum(dinvp * (a0b[...] + a1b[...] + ub[...]) + b1b_r[...],
                     0.0)
    h2 = (jnp.dot(ha, w2a_r[...], preferred_element_type=jnp.float32)
          + jnp.dot(hb, w2b_r[...], preferred_element_type=jnp.float32))
    u2_r[...] = h2 * dinvp


def _dense3_body(a0, a1, u2, dp0, dp1, b2_r, kf_r, out_r):
    dinvp = _dinv_packed(dp0, dp1, kf_r[...])
    out_r[...] = dinvp * (a0[...] + a1[...] + u2[...]) + b2_r[...]


def kernel(x, edge_index, W1, b1, W2, b2):
    n, f0 = x.shape
    f1 = W1.shape[1]
    f2 = W2.shape[1]
    fh = f1 // 2
    e = edge_index.shape[1]

    rtot = -(-e // (NS * CHUNK))
    r0 = _round_up(int(rtot * 0.47), 2 * K)
    r1 = _round_up(rtot - r0, 2 * K)
    ep = NS * (r0 + r1) * CHUNK
    t = _round_up(n + 1, NS * 8)
    tpt = t // NS

    src = edge_index[0]
    dst = edge_index[1]
    pad = ep - e
    srcp = jnp.concatenate([src, jnp.zeros((pad,), jnp.int32)]
                           ).reshape(ep // CHUNK, CHUNK)
    dstp = jnp.concatenate([dst, jnp.full((pad,), n, jnp.int32)]
                           ).reshape(ep // CHUNK, CHUNK)

    deg_call = pl.kernel(
        functools.partial(_deg_body, r0, r1, tpt, t),
        out_type=jax.ShapeDtypeStruct((NC * t,), jnp.float32),
        mesh=_sc_mesh(),
        scratch_types=[
            pltpu.VMEM_SHARED((t,), jnp.float32),
            pltpu.VMEM((CHUNK,), jnp.float32),
            pltpu.VMEM((K, CHUNK), jnp.int32),
            pltpu.VMEM((K, CHUNK), jnp.int32),
            pltpu.SemaphoreType.DMA,
            pltpu.VMEM((tpt,), jnp.float32),
        ],
        compiler_params=_SC_PARAMS,
    )
    degp = deg_call(dstp, jnp.ones((CHUNK,), jnp.float32),
                    jnp.zeros((tpt,), jnp.float32)).reshape(NC, t)

    pk = 128 // fh
    rows = t // pk
    dp0_pk = degp[0].reshape(rows, pk)
    dp1_pk = degp[1].reshape(rows, pk)
    eye = jnp.eye(pk, dtype=jnp.float32)
    kf = jnp.kron(eye, jnp.ones((1, fh), jnp.float32))
    wa = jnp.kron(eye, W1[:, :fh])
    wb = jnp.kron(eye, W1[:, fh:])
    w2a = jnp.kron(eye, W2[:fh])
    w2b = jnp.kron(eye, W2[fh:])
    b1ap = jnp.tile(b1[:fh], pk).reshape(1, 128)
    b1bp = jnp.tile(b1[fh:], pk).reshape(1, 128)
    b2p = jnp.tile(b2, pk).reshape(1, 128)
    xp = jnp.concatenate(
        [x.reshape(n * f0), jnp.zeros(((t - n) * f0,), jnp.float32)]
    ).reshape(rows, pk * f0)

    pkd = jax.ShapeDtypeStruct((rows, 128), jnp.float32)

    u1a, u1b = pl.pallas_call(
        _dense1_body, out_shape=[pkd, pkd],
    )(dp0_pk, dp1_pk, xp, wa, wb, kf)

    layer_scratch = [
        pltpu.VMEM_SHARED((t, fh), jnp.float32),
        pltpu.VMEM((K, CHUNK), jnp.int32),
        pltpu.VMEM((K, CHUNK), jnp.int32),
        pltpu.VMEM((K, CHUNK, fh), jnp.float32),
        pltpu.VMEM((K, CHUNK), jnp.int32),
        pltpu.VMEM((K, CHUNK), jnp.int32),
        pltpu.VMEM((K, CHUNK, fh), jnp.float32),
        pltpu.SemaphoreType.DMA,
        pltpu.SemaphoreType.DMA,
        pltpu.VMEM((tpt // 2, fh), jnp.float32),
    ]
    z2 = jnp.zeros((tpt // 2, fh), jnp.float32)

    l1_call = pl.kernel(
        functools.partial(_layer1_body, r0, r1, tpt),
        out_type=[jax.ShapeDtypeStruct((NC, t, fh), jnp.float32),
                  jax.ShapeDtypeStruct((NC, t, fh), jnp.float32)],
        mesh=_sc_mesh(),
        scratch_types=layer_scratch,
        compiler_params=_SC_PARAMS,
    )
    acc1a, acc1b = l1_call(u1a.reshape(t, fh), u1b.reshape(t, fh),
                           srcp, dstp, z2)

    u2 = pl.pallas_call(
        _dense2_body, out_shape=pkd,
    )(acc1a[0].reshape(rows, 128), acc1a[1].reshape(rows, 128),
      acc1b[0].reshape(rows, 128), acc1b[1].reshape(rows, 128),
      u1a, u1b, dp0_pk, dp1_pk, b1ap, b1bp, w2a, w2b, kf)

    l2_call = pl.kernel(
        functools.partial(_layer2_body, r0, r1, tpt),
        out_type=jax.ShapeDtypeStruct((NC, t, f2), jnp.float32),
        mesh=_sc_mesh(),
        scratch_types=layer_scratch,
        compiler_params=_SC_PARAMS,
    )
    acc2 = l2_call(u2.reshape(t, f2), srcp, dstp, z2)

    outp = pl.pallas_call(
        _dense3_body, out_shape=pkd,
    )(acc2[0].reshape(rows, 128), acc2[1].reshape(rows, 128),
      u2, dp0_pk, dp1_pk, b2p, kf)
    return outp.reshape(t * f2)[: n * f2].reshape(n, f2)


# speedup vs baseline: 1.2260x; 1.2260x over previous
import functools

import jax
import jax.numpy as jnp
from jax import lax
from jax.experimental import pallas as pl
from jax.experimental.pallas import tpu as pltpu
from jax.experimental.pallas import tpu_sc as plsc

NC = 2
NS = 16
NW = NC * NS
CHUNK = 128
K = 16


def _round_up(a, b):
    return (a + b - 1) // b * b


def _sc_mesh():
    return plsc.VectorSubcoreMesh(core_axis_name="c", subcore_axis_name="s",
                                  num_cores=NC, num_subcores=NS)


_SC_PARAMS = pltpu.CompilerParams(use_tc_tiling_on_sc=False)


def _worker_range(r0, r1, c, s):
    base = jnp.where(c == 0, s * r0, NS * r0 + s * r1)
    ngrp = jnp.where(c == 0, r0 // (2 * K), r1 // (2 * K))
    return base, ngrp


def _deg_body(r0, r1, tpt, t, dstr, ones_h, z1, degp, deg_sh, ones_v,
              didx_a, didx_b, ssem, zb):
    c = lax.axis_index("c")
    s = lax.axis_index("s")
    pltpu.sync_copy(z1, zb)
    pltpu.sync_copy(zb, deg_sh.at[pl.ds(s * tpt, tpt)])
    pltpu.sync_copy(ones_h, ones_v)
    plsc.subcore_barrier()
    base, ngrp = _worker_range(r0, r1, c, s)

    def grp(g2, carry):
        g = g2 * 2
        descs = []
        for didx, go in ((didx_a, g), (didx_b, g + 1)):
            pltpu.sync_copy(dstr.at[pl.ds(base + go * K, K)], didx)
            descs += [pltpu.async_copy(ones_v, deg_sh.at[didx.at[j]], ssem,
                                       add=True)
                      for j in range(K)]
        for d in descs:
            d.wait()
        return carry

    lax.fori_loop(0, ngrp, grp, 0)
    plsc.subcore_barrier()
    pltpu.sync_copy(deg_sh.at[pl.ds(s * tpt, tpt)], zb)
    pltpu.sync_copy(zb, degp.at[pl.ds(c * t + s * tpt, tpt)])


def _half_pass(r0, r1, tpt, c, s, u_h, srcr, dstr, z2, outp,
               tab_sh, bufs, gsem, ssem, zb):
    hpt = tpt // 2
    pltpu.sync_copy(z2, zb)
    for q in range(2):
        pltpu.sync_copy(zb, tab_sh.at[pl.ds(s * tpt + q * hpt, hpt)])
    plsc.subcore_barrier()
    base, ngrp = _worker_range(r0, r1, c, s)

    def grp(g2, carry):
        g = g2 * 2
        sdescs = []
        for (sidx, didx, rows), go in ((bufs[0], g), (bufs[1], g + 1)):
            pltpu.sync_copy(srcr.at[pl.ds(base + go * K, K)], sidx)
            pltpu.sync_copy(dstr.at[pl.ds(base + go * K, K)], didx)
            gdescs = [pltpu.async_copy(u_h.at[sidx.at[j]], rows.at[j], gsem)
                      for j in range(K)]
            for d in gdescs:
                d.wait()
            sdescs += [pltpu.async_copy(rows.at[j], tab_sh.at[didx.at[j]],
                                        ssem, add=True)
                       for j in range(K)]
        for d in sdescs:
            d.wait()
        return carry

    lax.fori_loop(0, ngrp, grp, 0)
    plsc.subcore_barrier()
    for q in range(2):
        pltpu.sync_copy(tab_sh.at[pl.ds(s * tpt + q * hpt, hpt)], zb)
        pltpu.sync_copy(zb, outp.at[c, pl.ds(s * tpt + q * hpt, hpt)])
    plsc.subcore_barrier()


def _layer1_body(r0, r1, tpt, ua, ub, srcr, dstr, z2, outa, outb,
                 tab_sh, sidx_a, didx_a, rows_a, sidx_b, didx_b, rows_b,
                 gsem, ssem, zb):
    c = lax.axis_index("c")
    s = lax.axis_index("s")
    bufs = ((sidx_a, didx_a, rows_a), (sidx_b, didx_b, rows_b))
    _half_pass(r0, r1, tpt, c, s, ua, srcr, dstr, z2, outa,
               tab_sh, bufs, gsem, ssem, zb)
    _half_pass(r0, r1, tpt, c, s, ub, srcr, dstr, z2, outb,
               tab_sh, bufs, gsem, ssem, zb)


def _layer2_body(r0, r1, tpt, u, srcr, dstr, z2, outp,
                 tab_sh, sidx_a, didx_a, rows_a, sidx_b, didx_b, rows_b,
                 gsem, ssem, zb):
    c = lax.axis_index("c")
    s = lax.axis_index("s")
    bufs = ((sidx_a, didx_a, rows_a), (sidx_b, didx_b, rows_b))
    _half_pass(r0, r1, tpt, c, s, u, srcr, dstr, z2, outp,
               tab_sh, bufs, gsem, ssem, zb)


def _dinv_packed(dp0, dp1, kf):
    deg = dp0[...] + dp1[...] + 1.0
    return jnp.dot(lax.rsqrt(deg), kf, preferred_element_type=jnp.float32)


def _dense1_body(dp0, dp1, xp, wa_r, wb_r, kf_r, ua_r, ub_r):
    dinvp = _dinv_packed(dp0, dp1, kf_r[...])
    ua_r[...] = jnp.dot(xp[...], wa_r[...],
                        preferred_element_type=jnp.float32) * dinvp
    ub_r[...] = jnp.dot(xp[...], wb_r[...],
                        preferred_element_type=jnp.float32) * dinvp


def _dense2_body(a0a, a1a, a0b, a1b, ua, ub, dp0, dp1, b1a_r, b1b_r,
                 w2a_r, w2b_r, kf_r, u2_r):
    dinvp = _dinv_packed(dp0, dp1, kf_r[...])
    ha = jnp.maximum(dinvp * (a0a[...] + a1a[...] + ua[...]) + b1a_r[...],
                     0.0)
    hb = jnp.maximum(dinvp * (a0b[...] + a1b[...] + ub[...]) + b1b_r[...],
                     0.0)
    h2 = (jnp.dot(ha, w2a_r[...], preferred_element_type=jnp.float32)
          + jnp.dot(hb, w2b_r[...], preferred_element_type=jnp.float32))
    u2_r[...] = h2 * dinvp


def _dense3_body(a0, a1, u2, dp0, dp1, b2_r, kf_r, out_r):
    dinvp = _dinv_packed(dp0, dp1, kf_r[...])
    out_r[...] = dinvp * (a0[...] + a1[...] + u2[...]) + b2_r[...]


def kernel(x, edge_index, W1, b1, W2, b2):
    n, f0 = x.shape
    f1 = W1.shape[1]
    f2 = W2.shape[1]
    fh = f1 // 2
    e = edge_index.shape[1]

    rtot = -(-e // (NS * CHUNK))
    r0 = _round_up(int(rtot * 0.47), 2 * K)
    r1 = _round_up(rtot - r0, 2 * K)
    ep = NS * (r0 + r1) * CHUNK
    t = _round_up(n + 1, NS * 8)
    tpt = t // NS

    src = edge_index[0]
    dst = edge_index[1]
    pad = ep - e
    srcp = jnp.concatenate([src, jnp.zeros((pad,), jnp.int32)]
                           ).reshape(ep // CHUNK, CHUNK)
    dstp = jnp.concatenate([dst, jnp.full((pad,), n, jnp.int32)]
                           ).reshape(ep // CHUNK, CHUNK)

    deg_call = pl.kernel(
        functools.partial(_deg_body, r0, r1, tpt, t),
        out_type=jax.ShapeDtypeStruct((NC * t,), jnp.float32),
        mesh=_sc_mesh(),
        scratch_types=[
            pltpu.VMEM_SHARED((t,), jnp.float32),
            pltpu.VMEM((CHUNK,), jnp.float32),
            pltpu.VMEM((K, CHUNK), jnp.int32),
            pltpu.VMEM((K, CHUNK), jnp.int32),
            pltpu.SemaphoreType.DMA,
            pltpu.VMEM((tpt,), jnp.float32),
        ],
        compiler_params=_SC_PARAMS,
    )
    degp = deg_call(dstp, jnp.ones((CHUNK,), jnp.float32),
                    jnp.zeros((tpt,), jnp.float32)).reshape(NC, t)

    pk = 128 // fh
    rows = t // pk
    dp0_pk = degp[0].reshape(rows, pk)
    dp1_pk = degp[1].reshape(rows, pk)
    eye = jnp.eye(pk, dtype=jnp.float32)
    kf = jnp.kron(eye, jnp.ones((1, fh), jnp.float32))
    wa = jnp.kron(eye, W1[:, :fh])
    wb = jnp.kron(eye, W1[:, fh:])
    w2a = jnp.kron(eye, W2[:fh])
    w2b = jnp.kron(eye, W2[fh:])
    b1ap = jnp.tile(b1[:fh], pk).reshape(1, 128)
    b1bp = jnp.tile(b1[fh:], pk).reshape(1, 128)
    b2p = jnp.tile(b2, pk).reshape(1, 128)
    xp = jnp.concatenate(
        [x.reshape(n * f0), jnp.zeros(((t - n) * f0,), jnp.float32)]
    ).reshape(rows, pk * f0)

    pkd = jax.ShapeDtypeStruct((rows, 128), jnp.float32)

    u1a, u1b = pl.pallas_call(
        _dense1_body, out_shape=[pkd, pkd],
    )(dp0_pk, dp1_pk, xp, wa, wb, kf)

    layer_scratch = [
        pltpu.VMEM_SHARED((t, fh), jnp.float32),
        pltpu.VMEM((K, CHUNK), jnp.int32),
        pltpu.VMEM((K, CHUNK), jnp.int32),
        pltpu.VMEM((K, CHUNK, fh), jnp.float32),
        pltpu.VMEM((K, CHUNK), jnp.int32),
        pltpu.VMEM((K, CHUNK), jnp.int32),
        pltpu.VMEM((K, CHUNK, fh), jnp.float32),
        pltpu.SemaphoreType.DMA,
        pltpu.SemaphoreType.DMA,
        pltpu.VMEM((tpt // 2, fh), jnp.float32),
    ]
    z2 = jnp.zeros((tpt // 2, fh), jnp.float32)

    l1_call = pl.kernel(
        functools.partial(_layer1_body, r0, r1, tpt),
        out_type=[jax.ShapeDtypeStruct((NC, t, fh), jnp.float32),
                  jax.ShapeDtypeStruct((NC, t, fh), jnp.float32)],
        mesh=_sc_mesh(),
        scratch_types=layer_scratch,
        compiler_params=_SC_PARAMS,
    )
    acc1a, acc1b = l1_call(u1a.reshape(t, fh), u1b.reshape(t, fh),
                           srcp, dstp, z2)

    u2 = pl.pallas_call(
        _dense2_body, out_shape=pkd,
    )(acc1a[0].reshape(rows, 128), acc1a[1].reshape(rows, 128),
      acc1b[0].reshape(rows, 128), acc1b[1].reshape(rows, 128),
      u1a, u1b, dp0_pk, dp1_pk, b1ap, b1bp, w2a, w2b, kf)

    l2_call = pl.kernel(
        functools.partial(_layer2_body, r0, r1, tpt),
        out_type=jax.ShapeDtypeStruct((NC, t, f2), jnp.float32),
        mesh=_sc_mesh(),
        scratch_types=layer_scratch,
        compiler_params=_SC_PARAMS,
    )
    acc2 = l2_call(u2.reshape(t, f2), srcp, dstp, z2)

    outp = pl.pallas_call(
        _dense3_body, out_shape=pkd,
    )(acc2[0].reshape(rows, 128), acc2[1].reshape(rows, 128),
      u2, dp0_pk, dp1_pk, b2p, kf)
    return outp.reshape(t * f2)[: n * f2].reshape(n, f2)
